# Initial kernel scaffold; baseline (speedup 1.0000x reference)
import jax
import jax.numpy as jnp
from jax.experimental import pallas as pl


def kernel(x, mask):
    raise NotImplementedError("write your pallas kernel here")


# speedup vs baseline: 6.5330x; 6.5330x over previous
import jax
import jax.numpy as jnp
from jax.experimental import pallas as pl
from jax.experimental.pallas import tpu as pltpu

_B, _C, _T = 16, 128, 4096
_T2 = _T // 2
_NOUT = _T2 // 128
_RPB = 2


def _unzip_chunk(c0, c1, idx_e, lane_lt64):
    ge0 = jnp.take_along_axis(c0, idx_e, axis=1, mode="promise_in_bounds")
    ge1 = jnp.take_along_axis(c1, idx_e, axis=1, mode="promise_in_bounds")
    go0 = jnp.take_along_axis(c0, idx_e + 1, axis=1, mode="promise_in_bounds")
    go1 = jnp.take_along_axis(c1, idx_e + 1, axis=1, mode="promise_in_bounds")
    return jnp.where(lane_lt64, ge0, ge1), jnp.where(lane_lt64, go0, go1)


def _resize_kernel(maskf_ref, x_ref, out_ref, moutf_ref):
    b = pl.program_id(0)

    idx_e_m = (2 * jax.lax.broadcasted_iota(jnp.int32, (_B, 128), 1)) & 127
    lane_lt64_m = jax.lax.broadcasted_iota(jnp.int32, (_B, 128), 1) < 64

    @pl.when(b == 0)
    def _mask_out():
        mall = maskf_ref[...]
        mp = mall + pltpu.roll(mall, _T - 1, 1)
        mov = jnp.where(mp > 1.5, 1.0, 0.0)
        for m in range(_NOUT):
            c0 = mov[:, 256 * m:256 * m + 128]
            c1 = mov[:, 256 * m + 128:256 * m + 256]
            ev, _ = _unzip_chunk(c0, c1, idx_e_m, lane_lt64_m)
            moutf_ref[:, pl.ds(128 * m, 128)] = ev

    for r in range(_RPB):
        _resize_row(maskf_ref, x_ref, out_ref, _RPB * b + r, r)


def _resize_row(maskf_ref, x_ref, out_ref, row, r):
    mrow = maskf_ref[pl.ds(row, 1), :]
    Lf = float(_T) - jnp.sum(mrow)
    ti = jax.lax.broadcasted_iota(jnp.int32, (1, _T), 1)
    mp_row = mrow + pltpu.roll(mrow, _T - 1, 1)
    pad_pairs = jnp.where(
        jnp.logical_and(mp_row > 1.5, (ti & 1) == 0), 1.0, 0.0)
    Of = float(_T2) - jnp.sum(pad_pairs)
    scale = Lf / jnp.maximum(Of, 1.0)

    ji = jax.lax.broadcasted_iota(jnp.int32, (1, _T2), 1)
    j = ji.astype(jnp.float32)
    src = (j + 0.5) * scale - 0.5
    src = jnp.clip(src, 0.0, Lf - 1.0)
    lof = jnp.floor(src)
    w = src - lof
    lo = lof.astype(jnp.int32)
    hi = jnp.minimum(lo + 1, Lf.astype(jnp.int32) - 1)
    keep = j < Of
    w1 = jnp.where(keep, 1.0 - w, 0.0)
    w2 = jnp.where(keep, w, 0.0)

    t0 = 2 * ji
    lidx = (t0 & 255) + jnp.where(lo == t0, 1, 0)
    sel0 = lidx < 128
    gidx = lidx & 127

    xb = x_ref[r]
    xs = pltpu.roll(xb, 1, 1)
    for m in range(_NOUT):
        c0 = xb[:, 256 * m:256 * m + 128]
        c1 = xb[:, 256 * m + 128:256 * m + 256]
        s0c = xs[:, 256 * m:256 * m + 128]
        s1c = xs[:, 256 * m + 128:256 * m + 256]
        s0, s1 = 128 * m, 128 * (m + 1)
        gi = jnp.broadcast_to(gidx[:, s0:s1], (_C, 128))
        sm = jnp.broadcast_to(sel0[:, s0:s1], (_C, 128))
        glo0 = jnp.take_along_axis(s0c, gi, axis=1, mode="promise_in_bounds")
        glo1 = jnp.take_along_axis(s1c, gi, axis=1, mode="promise_in_bounds")
        ghi0 = jnp.take_along_axis(c0, gi, axis=1, mode="promise_in_bounds")
        ghi1 = jnp.take_along_axis(c1, gi, axis=1, mode="promise_in_bounds")
        xlo = jnp.where(sm, glo0, glo1)
        xhi = jnp.where(sm, ghi0, ghi1)
        out_ref[pl.ds(r, 1), :, pl.ds(s0, 128)] = (
            xlo * w1[:, s0:s1] + xhi * w2[:, s0:s1]
        ).reshape(1, _C, 128)


def kernel(x, mask):
    maskf = mask.astype(jnp.float32)
    out, moutf = pl.pallas_call(
        _resize_kernel,
        grid=(_B // _RPB,),
        in_specs=[
            pl.BlockSpec((_B, _T), lambda b: (0, 0)),
            pl.BlockSpec((_RPB, _C, _T), lambda b: (b, 0, 0)),
        ],
        out_specs=[
            pl.BlockSpec((_RPB, _C, _T2), lambda b: (b, 0, 0)),
            pl.BlockSpec((_B, _T2), lambda b: (0, 0)),
        ],
        out_shape=[
            jax.ShapeDtypeStruct((_B, _C, _T2), jnp.float32),
            jax.ShapeDtypeStruct((_B, _T2), jnp.float32),
        ],
    )(maskf, x)
    return out, moutf.astype(jnp.bool_)
